# baseline (speedup 1.0000x reference)
import functools

import jax
import jax.numpy as jnp
from jax import lax
from jax.experimental import pallas as pl
from jax.experimental.pallas import tpu as pltpu
from jax.experimental.pallas import tpu_sc as plsc

N = 50000
E = 800000
G = 64

NC = 2
NS = 16
NW = NC * NS

NPAD = 50176
NBLK = 49
BLK = 1024
EPAD = 819200
EW = EPAD // NW
ET = EPAD // NS
CHA = 6400
CHB = 3200
NSLOT = 1
F1 = 128
ZROWS = 56
F2 = 256
NRANGE = 4
R = NPAD // NRANGE
STRIPE = R // NS
BATCH = 128

@functools.lru_cache(maxsize=None)
def _sc_mesh():
    return plsc.VectorSubcoreMesh(core_axis_name="c", subcore_axis_name="s",
                                  num_cores=NC, num_subcores=NS)


def _zero_vmem_1d(ref, n):
    z = jnp.zeros((16,), jnp.float32)

    def body(i, _):
        ref[pl.ds(i * 16, 16)] = z
        return 0

    lax.fori_loop(0, n // 16, body, 0)


def _edge_scalar_body(with_gather, src_hbm, dst_hbm, q_hbm, out_hbm,
                      qv, accv, srcb, dstb):
    c = lax.axis_index("c")
    s = lax.axis_index("s")
    w = s * NC + c
    _zero_vmem_1d(accv, NPAD)
    if with_gather:
        pltpu.sync_copy(q_hbm, qv)
    base_w = w * EW
    ones = jnp.ones((16,), jnp.float32)

    def chunk(ci, _):
        cb = base_w + ci * CHA
        pltpu.sync_copy(dst_hbm.at[pl.ds(cb, CHA)], dstb)
        if with_gather:
            pltpu.sync_copy(src_hbm.at[pl.ds(cb, CHA)], srcb)

        def edge(j, _):
            d16 = dstb[pl.ds(j * 16, 16)]
            if with_gather:
                s16 = srcb[pl.ds(j * 16, 16)]
                vals = plsc.load_gather(qv, [s16])
            else:
                vals = ones
            plsc.addupdate_scatter(accv, [d16], vals)
            return 0

        lax.fori_loop(0, CHA // 16, edge, 0)
        return 0

    lax.fori_loop(0, EW // CHA, chunk, 0)
    pltpu.sync_copy(accv, out_hbm.at[w])


@functools.lru_cache(maxsize=None)
def _pass_deg():
    @functools.partial(
        pl.kernel,
        out_type=jax.ShapeDtypeStruct((NW, NPAD), jnp.float32),
        mesh=_sc_mesh(),
        compiler_params=pltpu.CompilerParams(needs_layout_passes=False),
        scratch_types=[
            pltpu.VMEM((NPAD,), jnp.float32),
            pltpu.VMEM((CHA,), jnp.int32),
        ],
    )
    def body(dst_hbm, out_hbm, accv, dstb):
        _edge_scalar_body(False, None, dst_hbm, None, out_hbm,
                          None, accv, None, dstb)

    return body


@functools.lru_cache(maxsize=None)
def _pass_u():
    @functools.partial(
        pl.kernel,
        out_type=jax.ShapeDtypeStruct((NW, NPAD), jnp.float32),
        mesh=_sc_mesh(),
        compiler_params=pltpu.CompilerParams(needs_layout_passes=False),
        scratch_types=[
            pltpu.VMEM((NPAD,), jnp.float32),
            pltpu.VMEM((NPAD,), jnp.float32),
            pltpu.VMEM((CHA,), jnp.int32),
            pltpu.VMEM((CHA,), jnp.int32),
        ],
    )
    def body(src_hbm, dst_hbm, q_hbm, out_hbm, qv, accv, srcb, dstb):
        _edge_scalar_body(True, src_hbm, dst_hbm, q_hbm, out_hbm,
                          qv, accv, srcb, dstb)

    return body


def _pass_t_decorator(fn):
    return functools.partial(
        pl.kernel,
        out_type=jax.ShapeDtypeStruct((NPAD, F1), jnp.float32),
        mesh=_sc_mesh(),
        compiler_params=pltpu.CompilerParams(needs_layout_passes=False),
        scratch_types=[
        pltpu.VMEM_SHARED((R + 16, F1), jnp.float32),
        pltpu.VMEM((CHB,), jnp.int32),
        pltpu.VMEM((CHB,), jnp.int32),
        pltpu.VMEM((CHB + BATCH + 16,), jnp.int32),
        pltpu.VMEM((CHB + BATCH + 16,), jnp.int32),
        pltpu.VMEM((NSLOT, BATCH), jnp.int32),
        pltpu.VMEM((NSLOT, BATCH, F1), jnp.float32),
        pltpu.SemaphoreType.DMA,
        pltpu.SemaphoreType.DMA,
    ],
    )(fn)


@functools.lru_cache(maxsize=None)
def _pass_t():
    return _pass_t_decorator(_pass_t_impl)


def _pass_t_impl(src_hbm, dst_hbm, p_hbm, t_hbm,
                 accsh, srcb, dstb, msrc, mdst, brow, stage,
                 semg, semsc):
    c = lax.axis_index("c")
    s = lax.axis_index("s")
    z16 = jnp.zeros((16,), jnp.float32)
    sent_src = jnp.full((16,), N, jnp.int32)
    sent_dst = jnp.full((16,), R, jnp.int32)

    for ph in range(NRANGE // NC):
        r_idx = ph * NC + c
        rbase = r_idx * R
        def zrow(rr, _):
            for k in range(F1 // 16):
                stage[0, rr, pl.ds(k * 16, 16)] = z16
            return 0

        lax.fori_loop(0, BATCH, zrow, 0)
        for z in range(STRIPE // BATCH):
            pltpu.sync_copy(stage.at[0],
                            accsh.at[pl.ds(s * STRIPE + z * BATCH, BATCH)])
        rem = STRIPE % BATCH
        if rem:
            pltpu.sync_copy(
                stage.at[0, pl.ds(0, rem)],
                accsh.at[pl.ds(s * STRIPE + (STRIPE // BATCH) * BATCH, rem)])
        plsc.subcore_barrier()

        def chunk(ci, _):
            cb = s * ET + ci * CHB
            pltpu.sync_copy(src_hbm.at[pl.ds(cb, CHB)], srcb)
            pltpu.sync_copy(dst_hbm.at[pl.ds(cb, CHB)], dstb)

            def filt(j, cnt):
                d16 = dstb[pl.ds(j * 16, 16)]
                s16 = srcb[pl.ds(j * 16, 16)]
                dloc = d16 - rbase
                m = (dloc >= 0) & (dloc < R)
                plsc.store_compressed(msrc.at[pl.ds(cnt, 16)], s16, mask=m)
                plsc.store_compressed(mdst.at[pl.ds(cnt, 16)], dloc, mask=m)
                return cnt + jnp.sum(m.astype(jnp.int32))

            mcount = lax.fori_loop(0, CHB // 16, filt, 0)
            for k in range(BATCH // 16):
                msrc[pl.ds(mcount + k * 16, 16)] = sent_src
                mdst[pl.ds(mcount + k * 16, 16)] = sent_dst
            nb = (mcount + BATCH - 1) // BATCH
            ngf = nb // NSLOT

            def group(g, _):
                gdescs = []
                for b in range(NSLOT):
                    base = (g * NSLOT + b) * BATCH
                    for k in range(BATCH // 16):
                        brow[b, pl.ds(k * 16, 16)] = (
                            mdst[pl.ds(base + k * 16, 16)])
                    gdescs.append(pltpu.async_copy(
                        p_hbm.at[msrc.at[pl.ds(base, BATCH)]],
                        stage.at[b], semg))
                sdescs = []
                for b in range(NSLOT):
                    gdescs[b].wait()
                    sdescs.append(pltpu.async_copy(
                        stage.at[b], accsh.at[brow.at[b]], semsc, add=True))
                for d in sdescs:
                    d.wait()
                return 0

            lax.fori_loop(0, ngf, group, 0)
            for b in range(NSLOT):
                j = ngf * NSLOT + b

                @pl.when(j < nb)
                def _tail(b=b, j=j):
                    base = j * BATCH
                    for k in range(BATCH // 16):
                        brow[b, pl.ds(k * 16, 16)] = (
                            mdst[pl.ds(base + k * 16, 16)])
                    pltpu.async_copy(
                        p_hbm.at[msrc.at[pl.ds(base, BATCH)]],
                        stage.at[b], semg).wait()
                    pltpu.sync_copy(stage.at[b], accsh.at[brow.at[b]],
                                    add=True)
            return 0

        lax.fori_loop(0, ET // CHB, chunk, 0)
        plsc.subcore_barrier()
        pltpu.sync_copy(
            accsh.at[pl.ds(s * STRIPE, STRIPE)],
            t_hbm.at[pl.ds(rbase + s * STRIPE, STRIPE)])
        plsc.subcore_barrier()


def _silu(x):
    return x * jax.nn.sigmoid(x)


def _stage1_body(degp_ref, x_ref, dinv_ref, q_ref):
    i = pl.program_id(0)
    deg = 1.0 + jnp.sum(degp_ref[...], axis=-1, keepdims=True)
    gid = i * BLK + lax.broadcasted_iota(jnp.int32, (BLK, 1), 0)
    valid = gid < N
    dinv = jnp.where(valid, lax.rsqrt(deg), 0.0)
    dinv_ref[...] = dinv
    q_ref[...] = dinv * x_ref[...]


def _stage2_body(up_ref, q_ref, dinv_ref, w1_ref, b1_ref, p_ref):
    i = pl.program_id(0)
    dinv = dinv_ref[...]
    u = dinv * (jnp.sum(up_ref[...], axis=-1, keepdims=True) + q_ref[...])
    h1 = _silu(u * w1_ref[...] + b1_ref[...])
    gid = i * BLK + lax.broadcasted_iota(jnp.int32, (BLK, 1), 0)
    p_ref[...] = jnp.where(gid < N, dinv * h1, 0.0)


def _stage3_body(t_ref, p_ref, dinv_ref, bat_ref, w2_ref, b2_ref,
                 wl1_ref, bl1_ref, wl2_ref, bl2_ref, out_ref, gacc, cacc):
    i = pl.program_id(0)

    @pl.when(i == 0)
    def _():
        gacc[...] = jnp.zeros_like(gacc)
        cacc[...] = jnp.zeros_like(cacc)

    agg = dinv_ref[...] * (t_ref[...] + p_ref[...])
    out2 = jnp.dot(agg, w2_ref[...],
                   preferred_element_type=jnp.float32) + b2_ref[...]
    h2 = _silu(out2)
    oh = (bat_ref[...] ==
          lax.broadcasted_iota(jnp.int32, (1, G), 1)).astype(jnp.float32)
    gacc[...] += lax.dot_general(oh, h2, (((0,), (0,)), ((), ())),
                                 preferred_element_type=jnp.float32)
    cacc[...] += lax.dot_general(oh, jnp.ones((BLK, 1), jnp.float32),
                                 (((0,), (0,)), ((), ())),
                                 preferred_element_type=jnp.float32)

    @pl.when(i == NBLK - 1)
    def _():
        g = gacc[...] / jnp.maximum(cacc[...], 1.0)
        z1 = _silu(jnp.dot(g, wl1_ref[...],
                           preferred_element_type=jnp.float32) + bl1_ref[...])
        out_ref[...] = jnp.dot(z1, wl2_ref[...],
                               preferred_element_type=jnp.float32) + bl2_ref[...]


def _stage1(degp_t, x1):
    return pl.pallas_call(
        _stage1_body,
        grid=(NBLK,),
        in_specs=[
            pl.BlockSpec((BLK, NW), lambda i: (i, 0)),
            pl.BlockSpec((BLK, 1), lambda i: (i, 0)),
        ],
        out_specs=[
            pl.BlockSpec((BLK, 1), lambda i: (i, 0)),
            pl.BlockSpec((BLK, 1), lambda i: (i, 0)),
        ],
        out_shape=[
            jax.ShapeDtypeStruct((NPAD, 1), jnp.float32),
            jax.ShapeDtypeStruct((NPAD, 1), jnp.float32),
        ],
    )(degp_t, x1)


def _stage2(up_t, qc, dinvc, w1p, b1p):
    return pl.pallas_call(
        _stage2_body,
        grid=(NBLK,),
        in_specs=[
            pl.BlockSpec((BLK, NW), lambda i: (i, 0)),
            pl.BlockSpec((BLK, 1), lambda i: (i, 0)),
            pl.BlockSpec((BLK, 1), lambda i: (i, 0)),
            pl.BlockSpec((1, F1), lambda i: (0, 0)),
            pl.BlockSpec((1, F1), lambda i: (0, 0)),
        ],
        out_specs=pl.BlockSpec((BLK, F1), lambda i: (i, 0)),
        out_shape=jax.ShapeDtypeStruct((NPAD, F1), jnp.float32),
    )(up_t, qc, dinvc, w1p, b1p)


def _stage3(t, p, dinvc, batc, w2p, b2p, wl1p, bl1p, wl2p, bl2p):
    return pl.pallas_call(
        _stage3_body,
        grid=(NBLK,),
        in_specs=[
            pl.BlockSpec((BLK, F1), lambda i: (i, 0)),
            pl.BlockSpec((BLK, F1), lambda i: (i, 0)),
            pl.BlockSpec((BLK, 1), lambda i: (i, 0)),
            pl.BlockSpec((BLK, 1), lambda i: (i, 0)),
            pl.BlockSpec((F1, F2), lambda i: (0, 0)),
            pl.BlockSpec((1, F2), lambda i: (0, 0)),
            pl.BlockSpec((F2, 128), lambda i: (0, 0)),
            pl.BlockSpec((1, 128), lambda i: (0, 0)),
            pl.BlockSpec((128, 128), lambda i: (0, 0)),
            pl.BlockSpec((1, 128), lambda i: (0, 0)),
        ],
        out_specs=pl.BlockSpec((G, 128), lambda i: (0, 0)),
        out_shape=jax.ShapeDtypeStruct((G, 128), jnp.float32),
        scratch_shapes=[
            pltpu.VMEM((G, F2), jnp.float32),
            pltpu.VMEM((G, 1), jnp.float32),
        ],
    )(t, p, dinvc, batc, w2p, b2p, wl1p, bl1p, wl2p, bl2p)


def kernel(x, edge_index, batch, W1, b1, W2, b2, Wl1, bl1, Wl2, bl2):
    f32 = jnp.float32
    src = jnp.pad(edge_index[0], (0, EPAD - E), constant_values=N)
    dst = jnp.pad(edge_index[1], (0, EPAD - E), constant_values=N)
    x1 = jnp.pad(x.astype(f32), ((0, NPAD - N), (0, 0)))
    batc = jnp.pad(batch, (0, NPAD - N), constant_values=G)[:, None]

    w1p = jnp.pad(W1.astype(f32), ((0, 0), (0, F1 - 100)))
    b1p = jnp.pad(b1.astype(f32), (0, F1 - 100))[None, :]
    w2p = jnp.pad(W2.astype(f32), ((0, F1 - 100), (0, F2 - 200)))
    b2p = jnp.pad(b2.astype(f32), (0, F2 - 200))[None, :]
    wl1p = jnp.pad(Wl1.astype(f32), ((0, F2 - 200), (0, 128 - 100)))
    bl1p = jnp.pad(bl1.astype(f32), (0, 128 - 100))[None, :]
    wl2p = jnp.pad(Wl2.astype(f32), ((0, 128 - 100), (0, 127)))
    bl2p = jnp.pad(bl2.astype(f32), (0, 127))[None, :]

    degp = _pass_deg()(dst)
    dinvc, qc = _stage1(degp.T, x1)
    q1 = qc[:, 0]
    urawp = _pass_u()(src, dst, q1)
    p = _stage2(urawp.T, qc, dinvc, w1p, b1p)
    t = _pass_t()(src, dst, p)
    out = _stage3(t, p, dinvc, batc, w2p, b2p, wl1p, bl1p, wl2p, bl2p)
    return out[:, 0]


# speedup vs baseline: 1.0325x; 1.0325x over previous
import functools

import jax
import jax.numpy as jnp
from jax import lax
from jax.experimental import pallas as pl
from jax.experimental.pallas import tpu as pltpu
from jax.experimental.pallas import tpu_sc as plsc

N = 50000
E = 800000
G = 64

NC = 2
NS = 16
NW = NC * NS

NPAD = 50176
NBLK = 49
BLK = 1024
EPAD = 819200
EW = EPAD // NW
ET = EPAD // NS
CHA = 6400
CHB = 6400
NSLOT = 3
F1 = 128
ZROWS = 56
F2 = 256
NRANGE = 8
R = NPAD // NRANGE
STRIPE = R // NS
BATCH = 128

@functools.lru_cache(maxsize=None)
def _sc_mesh():
    return plsc.VectorSubcoreMesh(core_axis_name="c", subcore_axis_name="s",
                                  num_cores=NC, num_subcores=NS)


def _zero_vmem_1d(ref, n):
    z = jnp.zeros((16,), jnp.float32)

    def body(i, _):
        ref[pl.ds(i * 16, 16)] = z
        return 0

    lax.fori_loop(0, n // 16, body, 0)


def _edge_scalar_body(with_gather, src_hbm, dst_hbm, q_hbm, out_hbm,
                      qv, accv, srcb, dstb):
    c = lax.axis_index("c")
    s = lax.axis_index("s")
    w = s * NC + c
    _zero_vmem_1d(accv, NPAD)
    if with_gather:
        pltpu.sync_copy(q_hbm, qv)
    base_w = w * EW
    ones = jnp.ones((16,), jnp.float32)

    def chunk(ci, _):
        cb = base_w + ci * CHA
        pltpu.sync_copy(dst_hbm.at[pl.ds(cb, CHA)], dstb)
        if with_gather:
            pltpu.sync_copy(src_hbm.at[pl.ds(cb, CHA)], srcb)

        def edge(j, _):
            d16 = dstb[pl.ds(j * 16, 16)]
            if with_gather:
                s16 = srcb[pl.ds(j * 16, 16)]
                vals = plsc.load_gather(qv, [s16])
            else:
                vals = ones
            plsc.addupdate_scatter(accv, [d16], vals)
            return 0

        lax.fori_loop(0, CHA // 16, edge, 0)
        return 0

    lax.fori_loop(0, EW // CHA, chunk, 0)
    pltpu.sync_copy(accv, out_hbm.at[w])


@functools.lru_cache(maxsize=None)
def _pass_deg():
    @functools.partial(
        pl.kernel,
        out_type=jax.ShapeDtypeStruct((NW, NPAD), jnp.float32),
        mesh=_sc_mesh(),
        compiler_params=pltpu.CompilerParams(needs_layout_passes=False),
        scratch_types=[
            pltpu.VMEM((NPAD,), jnp.float32),
            pltpu.VMEM((CHA,), jnp.int32),
        ],
    )
    def body(dst_hbm, out_hbm, accv, dstb):
        _edge_scalar_body(False, None, dst_hbm, None, out_hbm,
                          None, accv, None, dstb)

    return body


@functools.lru_cache(maxsize=None)
def _pass_u():
    @functools.partial(
        pl.kernel,
        out_type=jax.ShapeDtypeStruct((NW, NPAD), jnp.float32),
        mesh=_sc_mesh(),
        compiler_params=pltpu.CompilerParams(needs_layout_passes=False),
        scratch_types=[
            pltpu.VMEM((NPAD,), jnp.float32),
            pltpu.VMEM((NPAD,), jnp.float32),
            pltpu.VMEM((CHA,), jnp.int32),
            pltpu.VMEM((CHA,), jnp.int32),
        ],
    )
    def body(src_hbm, dst_hbm, q_hbm, out_hbm, qv, accv, srcb, dstb):
        _edge_scalar_body(True, src_hbm, dst_hbm, q_hbm, out_hbm,
                          qv, accv, srcb, dstb)

    return body


def _pass_t_decorator(fn):
    return functools.partial(
        pl.kernel,
        out_type=jax.ShapeDtypeStruct((NPAD, F1), jnp.float32),
        mesh=_sc_mesh(),
        compiler_params=pltpu.CompilerParams(needs_layout_passes=False),
        scratch_types=[
        pltpu.VMEM_SHARED((R + 16, F1), jnp.float32),
        pltpu.VMEM((CHB,), jnp.int32),
        pltpu.VMEM((CHB,), jnp.int32),
        pltpu.VMEM((CHB + BATCH + 16,), jnp.int32),
        pltpu.VMEM((CHB + BATCH + 16,), jnp.int32),
        pltpu.VMEM((NSLOT, BATCH), jnp.int32),
        pltpu.VMEM((NSLOT, BATCH, F1), jnp.float32),
        pltpu.SemaphoreType.DMA,
        pltpu.SemaphoreType.DMA,
    ],
    )(fn)


@functools.lru_cache(maxsize=None)
def _pass_t():
    return _pass_t_decorator(_pass_t_impl)


def _pass_t_impl(src_hbm, dst_hbm, p_hbm, t_hbm,
                 accsh, srcb, dstb, msrc, mdst, brow, stage,
                 semg, semsc):
    c = lax.axis_index("c")
    s = lax.axis_index("s")
    z16 = jnp.zeros((16,), jnp.float32)
    sent_src = jnp.full((16,), N, jnp.int32)
    sent_dst = jnp.full((16,), R, jnp.int32)

    for ph in range(NRANGE // NC):
        r_idx = ph * NC + c
        rbase = r_idx * R
        def zrow(rr, _):
            for k in range(F1 // 16):
                stage[0, rr, pl.ds(k * 16, 16)] = z16
            return 0

        lax.fori_loop(0, BATCH, zrow, 0)
        for z in range(STRIPE // BATCH):
            pltpu.sync_copy(stage.at[0],
                            accsh.at[pl.ds(s * STRIPE + z * BATCH, BATCH)])
        rem = STRIPE % BATCH
        if rem:
            pltpu.sync_copy(
                stage.at[0, pl.ds(0, rem)],
                accsh.at[pl.ds(s * STRIPE + (STRIPE // BATCH) * BATCH, rem)])
        plsc.subcore_barrier()

        def chunk(ci, _):
            cb = s * ET + ci * CHB
            pltpu.sync_copy(src_hbm.at[pl.ds(cb, CHB)], srcb)
            pltpu.sync_copy(dst_hbm.at[pl.ds(cb, CHB)], dstb)

            def filt(j, cnt):
                d16 = dstb[pl.ds(j * 16, 16)]
                s16 = srcb[pl.ds(j * 16, 16)]
                dloc = d16 - rbase
                m = (dloc >= 0) & (dloc < R)
                plsc.store_compressed(msrc.at[pl.ds(cnt, 16)], s16, mask=m)
                plsc.store_compressed(mdst.at[pl.ds(cnt, 16)], dloc, mask=m)
                return cnt + jnp.sum(m.astype(jnp.int32))

            mcount = lax.fori_loop(0, CHB // 16, filt, 0)
            for k in range(BATCH // 16):
                msrc[pl.ds(mcount + k * 16, 16)] = sent_src
                mdst[pl.ds(mcount + k * 16, 16)] = sent_dst
            nb = (mcount + BATCH - 1) // BATCH
            ngf = nb // NSLOT

            def group(g, _):
                gdescs = []
                for b in range(NSLOT):
                    base = (g * NSLOT + b) * BATCH
                    for k in range(BATCH // 16):
                        brow[b, pl.ds(k * 16, 16)] = (
                            mdst[pl.ds(base + k * 16, 16)])
                    gdescs.append(pltpu.async_copy(
                        p_hbm.at[msrc.at[pl.ds(base, BATCH)]],
                        stage.at[b], semg))
                sdescs = []
                for b in range(NSLOT):
                    gdescs[b].wait()
                    sdescs.append(pltpu.async_copy(
                        stage.at[b], accsh.at[brow.at[b]], semsc, add=True))
                for d in sdescs:
                    d.wait()
                return 0

            lax.fori_loop(0, ngf, group, 0)
            for b in range(NSLOT):
                j = ngf * NSLOT + b

                @pl.when(j < nb)
                def _tail(b=b, j=j):
                    base = j * BATCH
                    for k in range(BATCH // 16):
                        brow[b, pl.ds(k * 16, 16)] = (
                            mdst[pl.ds(base + k * 16, 16)])
                    pltpu.async_copy(
                        p_hbm.at[msrc.at[pl.ds(base, BATCH)]],
                        stage.at[b], semg).wait()
                    pltpu.sync_copy(stage.at[b], accsh.at[brow.at[b]],
                                    add=True)
            return 0

        lax.fori_loop(0, ET // CHB, chunk, 0)
        plsc.subcore_barrier()
        pltpu.sync_copy(
            accsh.at[pl.ds(s * STRIPE, STRIPE)],
            t_hbm.at[pl.ds(rbase + s * STRIPE, STRIPE)])
        plsc.subcore_barrier()


def _silu(x):
    return x * jax.nn.sigmoid(x)


def _stage1_body(degp_ref, x_ref, dinv_ref, q_ref):
    i = pl.program_id(0)
    deg = 1.0 + jnp.sum(degp_ref[...], axis=-1, keepdims=True)
    gid = i * BLK + lax.broadcasted_iota(jnp.int32, (BLK, 1), 0)
    valid = gid < N
    dinv = jnp.where(valid, lax.rsqrt(deg), 0.0)
    dinv_ref[...] = dinv
    q_ref[...] = dinv * x_ref[...]


def _stage2_body(up_ref, q_ref, dinv_ref, w1_ref, b1_ref, p_ref):
    i = pl.program_id(0)
    dinv = dinv_ref[...]
    u = dinv * (jnp.sum(up_ref[...], axis=-1, keepdims=True) + q_ref[...])
    h1 = _silu(u * w1_ref[...] + b1_ref[...])
    gid = i * BLK + lax.broadcasted_iota(jnp.int32, (BLK, 1), 0)
    p_ref[...] = jnp.where(gid < N, dinv * h1, 0.0)


def _stage3_body(t_ref, p_ref, dinv_ref, bat_ref, w2_ref, b2_ref,
                 wl1_ref, bl1_ref, wl2_ref, bl2_ref, out_ref, gacc, cacc):
    i = pl.program_id(0)

    @pl.when(i == 0)
    def _():
        gacc[...] = jnp.zeros_like(gacc)
        cacc[...] = jnp.zeros_like(cacc)

    agg = dinv_ref[...] * (t_ref[...] + p_ref[...])
    out2 = jnp.dot(agg, w2_ref[...],
                   preferred_element_type=jnp.float32) + b2_ref[...]
    h2 = _silu(out2)
    oh = (bat_ref[...] ==
          lax.broadcasted_iota(jnp.int32, (1, G), 1)).astype(jnp.float32)
    gacc[...] += lax.dot_general(oh, h2, (((0,), (0,)), ((), ())),
                                 preferred_element_type=jnp.float32)
    cacc[...] += lax.dot_general(oh, jnp.ones((BLK, 1), jnp.float32),
                                 (((0,), (0,)), ((), ())),
                                 preferred_element_type=jnp.float32)

    @pl.when(i == NBLK - 1)
    def _():
        g = gacc[...] / jnp.maximum(cacc[...], 1.0)
        z1 = _silu(jnp.dot(g, wl1_ref[...],
                           preferred_element_type=jnp.float32) + bl1_ref[...])
        out_ref[...] = jnp.dot(z1, wl2_ref[...],
                               preferred_element_type=jnp.float32) + bl2_ref[...]


def _stage1(degp_t, x1):
    return pl.pallas_call(
        _stage1_body,
        grid=(NBLK,),
        in_specs=[
            pl.BlockSpec((BLK, NW), lambda i: (i, 0)),
            pl.BlockSpec((BLK, 1), lambda i: (i, 0)),
        ],
        out_specs=[
            pl.BlockSpec((BLK, 1), lambda i: (i, 0)),
            pl.BlockSpec((BLK, 1), lambda i: (i, 0)),
        ],
        out_shape=[
            jax.ShapeDtypeStruct((NPAD, 1), jnp.float32),
            jax.ShapeDtypeStruct((NPAD, 1), jnp.float32),
        ],
    )(degp_t, x1)


def _stage2(up_t, qc, dinvc, w1p, b1p):
    return pl.pallas_call(
        _stage2_body,
        grid=(NBLK,),
        in_specs=[
            pl.BlockSpec((BLK, NW), lambda i: (i, 0)),
            pl.BlockSpec((BLK, 1), lambda i: (i, 0)),
            pl.BlockSpec((BLK, 1), lambda i: (i, 0)),
            pl.BlockSpec((1, F1), lambda i: (0, 0)),
            pl.BlockSpec((1, F1), lambda i: (0, 0)),
        ],
        out_specs=pl.BlockSpec((BLK, F1), lambda i: (i, 0)),
        out_shape=jax.ShapeDtypeStruct((NPAD, F1), jnp.float32),
    )(up_t, qc, dinvc, w1p, b1p)


def _stage3(t, p, dinvc, batc, w2p, b2p, wl1p, bl1p, wl2p, bl2p):
    return pl.pallas_call(
        _stage3_body,
        grid=(NBLK,),
        in_specs=[
            pl.BlockSpec((BLK, F1), lambda i: (i, 0)),
            pl.BlockSpec((BLK, F1), lambda i: (i, 0)),
            pl.BlockSpec((BLK, 1), lambda i: (i, 0)),
            pl.BlockSpec((BLK, 1), lambda i: (i, 0)),
            pl.BlockSpec((F1, F2), lambda i: (0, 0)),
            pl.BlockSpec((1, F2), lambda i: (0, 0)),
            pl.BlockSpec((F2, 128), lambda i: (0, 0)),
            pl.BlockSpec((1, 128), lambda i: (0, 0)),
            pl.BlockSpec((128, 128), lambda i: (0, 0)),
            pl.BlockSpec((1, 128), lambda i: (0, 0)),
        ],
        out_specs=pl.BlockSpec((G, 128), lambda i: (0, 0)),
        out_shape=jax.ShapeDtypeStruct((G, 128), jnp.float32),
        scratch_shapes=[
            pltpu.VMEM((G, F2), jnp.float32),
            pltpu.VMEM((G, 1), jnp.float32),
        ],
    )(t, p, dinvc, batc, w2p, b2p, wl1p, bl1p, wl2p, bl2p)


def kernel(x, edge_index, batch, W1, b1, W2, b2, Wl1, bl1, Wl2, bl2):
    f32 = jnp.float32
    src = jnp.pad(edge_index[0], (0, EPAD - E), constant_values=N)
    dst = jnp.pad(edge_index[1], (0, EPAD - E), constant_values=N)
    x1 = jnp.pad(x.astype(f32), ((0, NPAD - N), (0, 0)))
    batc = jnp.pad(batch, (0, NPAD - N), constant_values=G)[:, None]

    w1p = jnp.pad(W1.astype(f32), ((0, 0), (0, F1 - 100)))
    b1p = jnp.pad(b1.astype(f32), (0, F1 - 100))[None, :]
    w2p = jnp.pad(W2.astype(f32), ((0, F1 - 100), (0, F2 - 200)))
    b2p = jnp.pad(b2.astype(f32), (0, F2 - 200))[None, :]
    wl1p = jnp.pad(Wl1.astype(f32), ((0, F2 - 200), (0, 128 - 100)))
    bl1p = jnp.pad(bl1.astype(f32), (0, 128 - 100))[None, :]
    wl2p = jnp.pad(Wl2.astype(f32), ((0, 128 - 100), (0, 127)))
    bl2p = jnp.pad(bl2.astype(f32), (0, 127))[None, :]

    degp = _pass_deg()(dst)
    dinvc, qc = _stage1(degp.T, x1)
    q1 = qc[:, 0]
    urawp = _pass_u()(src, dst, q1)
    p = _stage2(urawp.T, qc, dinvc, w1p, b1p)
    t = _pass_t()(src, dst, p)
    out = _stage3(t, p, dinvc, batc, w2p, b2p, wl1p, bl1p, wl2p, bl2p)
    return out[:, 0]


# speedup vs baseline: 1.3405x; 1.2982x over previous
import functools

import jax
import jax.numpy as jnp
from jax import lax
from jax.experimental import pallas as pl
from jax.experimental.pallas import tpu as pltpu
from jax.experimental.pallas import tpu_sc as plsc

N = 50000
E = 800000
G = 64

NC = 2
NS = 16
NW = NC * NS

NPAD = 50176
NBLK = 49
BLK = 1024
EPAD = 800768
EW = EPAD // NW
ET = EPAD // NS
CHA = 6256
CHB = 6256
NSLOT = 3
F1 = 128
ZROWS = 56
F2 = 256
NRANGE = 8
R = NPAD // NRANGE
STRIPE = R // NS
BATCH = 128

@functools.lru_cache(maxsize=None)
def _sc_mesh():
    return plsc.VectorSubcoreMesh(core_axis_name="c", subcore_axis_name="s",
                                  num_cores=NC, num_subcores=NS)


def _zero_vmem_1d(ref, n):
    z = jnp.zeros((16,), jnp.float32)

    def body(i, _):
        ref[pl.ds(i * 16, 16)] = z
        return 0

    lax.fori_loop(0, n // 16, body, 0)


def _edge_scalar_body(with_gather, src_hbm, dst_hbm, q_hbm, out_hbm,
                      qv, accv, srcb, dstb):
    c = lax.axis_index("c")
    s = lax.axis_index("s")
    w = s * NC + c
    _zero_vmem_1d(accv, NPAD)
    if with_gather:
        pltpu.sync_copy(q_hbm, qv)
    base_w = w * EW
    ones = jnp.ones((16,), jnp.float32)

    def chunk(ci, _):
        cb = base_w + ci * CHA
        pltpu.sync_copy(dst_hbm.at[pl.ds(cb, CHA)], dstb)
        if with_gather:
            pltpu.sync_copy(src_hbm.at[pl.ds(cb, CHA)], srcb)

        def edge(j, _):
            d16 = dstb[pl.ds(j * 16, 16)]
            if with_gather:
                s16 = srcb[pl.ds(j * 16, 16)]
                vals = plsc.load_gather(qv, [s16])
            else:
                vals = ones
            plsc.addupdate_scatter(accv, [d16], vals)
            return 0

        lax.fori_loop(0, CHA // 16, edge, 0)
        return 0

    lax.fori_loop(0, EW // CHA, chunk, 0)
    pltpu.sync_copy(accv, out_hbm.at[w])


@functools.lru_cache(maxsize=None)
def _pass_deg():
    @functools.partial(
        pl.kernel,
        out_type=jax.ShapeDtypeStruct((NW, NPAD), jnp.float32),
        mesh=_sc_mesh(),
        compiler_params=pltpu.CompilerParams(needs_layout_passes=False),
        scratch_types=[
            pltpu.VMEM((NPAD,), jnp.float32),
            pltpu.VMEM((CHA,), jnp.int32),
        ],
    )
    def body(dst_hbm, out_hbm, accv, dstb):
        _edge_scalar_body(False, None, dst_hbm, None, out_hbm,
                          None, accv, None, dstb)

    return body


@functools.lru_cache(maxsize=None)
def _pass_u():
    @functools.partial(
        pl.kernel,
        out_type=jax.ShapeDtypeStruct((NW, NPAD), jnp.float32),
        mesh=_sc_mesh(),
        compiler_params=pltpu.CompilerParams(needs_layout_passes=False),
        scratch_types=[
            pltpu.VMEM((NPAD,), jnp.float32),
            pltpu.VMEM((NPAD,), jnp.float32),
            pltpu.VMEM((CHA,), jnp.int32),
            pltpu.VMEM((CHA,), jnp.int32),
        ],
    )
    def body(src_hbm, dst_hbm, q_hbm, out_hbm, qv, accv, srcb, dstb):
        _edge_scalar_body(True, src_hbm, dst_hbm, q_hbm, out_hbm,
                          qv, accv, srcb, dstb)

    return body


def _pass_t_decorator(fn):
    return functools.partial(
        pl.kernel,
        out_type=jax.ShapeDtypeStruct((NPAD, F1), jnp.float32),
        mesh=_sc_mesh(),
        compiler_params=pltpu.CompilerParams(needs_layout_passes=False),
        scratch_types=[
        pltpu.VMEM_SHARED((R + 16, F1), jnp.float32),
        pltpu.VMEM((CHB,), jnp.int32),
        pltpu.VMEM((CHB,), jnp.int32),
        pltpu.VMEM((CHB + BATCH + 16,), jnp.int32),
        pltpu.VMEM((CHB + BATCH + 16,), jnp.int32),
        pltpu.VMEM((NSLOT, BATCH), jnp.int32),
        pltpu.VMEM((NSLOT, BATCH, F1), jnp.float32),
        pltpu.SemaphoreType.DMA,
        pltpu.SemaphoreType.DMA,
    ],
    )(fn)


@functools.lru_cache(maxsize=None)
def _pass_t():
    return _pass_t_decorator(_pass_t_impl)


def _pass_t_impl(src_hbm, dst_hbm, p_hbm, t_hbm,
                 accsh, srcb, dstb, msrc, mdst, brow, stage,
                 semg, semsc):
    c = lax.axis_index("c")
    s = lax.axis_index("s")
    z16 = jnp.zeros((16,), jnp.float32)
    sent_src = jnp.full((16,), N, jnp.int32)
    sent_dst = jnp.full((16,), R, jnp.int32)

    for ph in range(NRANGE // NC):
        r_idx = ph * NC + c
        rbase = r_idx * R
        def zrow(rr, _):
            for k in range(F1 // 16):
                stage[0, rr, pl.ds(k * 16, 16)] = z16
            return 0

        lax.fori_loop(0, BATCH, zrow, 0)
        for z in range(STRIPE // BATCH):
            pltpu.sync_copy(stage.at[0],
                            accsh.at[pl.ds(s * STRIPE + z * BATCH, BATCH)])
        rem = STRIPE % BATCH
        if rem:
            pltpu.sync_copy(
                stage.at[0, pl.ds(0, rem)],
                accsh.at[pl.ds(s * STRIPE + (STRIPE // BATCH) * BATCH, rem)])
        plsc.subcore_barrier()

        def chunk(ci, _):
            cb = s * ET + ci * CHB
            pltpu.sync_copy(src_hbm.at[pl.ds(cb, CHB)], srcb)
            pltpu.sync_copy(dst_hbm.at[pl.ds(cb, CHB)], dstb)

            def filt(j, cnt):
                d16 = dstb[pl.ds(j * 16, 16)]
                s16 = srcb[pl.ds(j * 16, 16)]
                dloc = d16 - rbase
                m = (dloc >= 0) & (dloc < R)
                plsc.store_compressed(msrc.at[pl.ds(cnt, 16)], s16, mask=m)
                plsc.store_compressed(mdst.at[pl.ds(cnt, 16)], dloc, mask=m)
                return cnt + jnp.sum(m.astype(jnp.int32))

            mcount = lax.fori_loop(0, CHB // 16, filt, 0)
            for k in range(BATCH // 16):
                msrc[pl.ds(mcount + k * 16, 16)] = sent_src
                mdst[pl.ds(mcount + k * 16, 16)] = sent_dst
            nb = (mcount + BATCH - 1) // BATCH
            ngf = nb // NSLOT

            def group(g, _):
                gdescs = []
                for b in range(NSLOT):
                    base = (g * NSLOT + b) * BATCH
                    for k in range(BATCH // 16):
                        brow[b, pl.ds(k * 16, 16)] = (
                            mdst[pl.ds(base + k * 16, 16)])
                    gdescs.append(pltpu.async_copy(
                        p_hbm.at[msrc.at[pl.ds(base, BATCH)]],
                        stage.at[b], semg))
                sdescs = []
                for b in range(NSLOT):
                    gdescs[b].wait()
                    sdescs.append(pltpu.async_copy(
                        stage.at[b], accsh.at[brow.at[b]], semsc, add=True))
                for d in sdescs:
                    d.wait()
                return 0

            lax.fori_loop(0, ngf, group, 0)
            for b in range(NSLOT):
                j = ngf * NSLOT + b

                @pl.when(j < nb)
                def _tail(b=b, j=j):
                    base = j * BATCH
                    for k in range(BATCH // 16):
                        brow[b, pl.ds(k * 16, 16)] = (
                            mdst[pl.ds(base + k * 16, 16)])
                    pltpu.async_copy(
                        p_hbm.at[msrc.at[pl.ds(base, BATCH)]],
                        stage.at[b], semg).wait()
                    pltpu.sync_copy(stage.at[b], accsh.at[brow.at[b]],
                                    add=True)
            return 0

        lax.fori_loop(0, ET // CHB, chunk, 0)
        plsc.subcore_barrier()
        pltpu.sync_copy(
            accsh.at[pl.ds(s * STRIPE, STRIPE)],
            t_hbm.at[pl.ds(rbase + s * STRIPE, STRIPE)])
        plsc.subcore_barrier()


def _silu(x):
    return x * jax.nn.sigmoid(x)


def _stage1_body(degp_ref, x_ref, dinv_ref, q_ref):
    i = pl.program_id(0)
    deg = 1.0 + jnp.sum(degp_ref[...], axis=-1, keepdims=True)
    gid = i * BLK + lax.broadcasted_iota(jnp.int32, (BLK, 1), 0)
    valid = gid < N
    dinv = jnp.where(valid, lax.rsqrt(deg), 0.0)
    dinv_ref[...] = dinv
    q_ref[...] = dinv * x_ref[...]


def _stage2_body(up_ref, q_ref, dinv_ref, w1_ref, b1_ref, p_ref):
    i = pl.program_id(0)
    dinv = dinv_ref[...]
    u = dinv * (jnp.sum(up_ref[...], axis=-1, keepdims=True) + q_ref[...])
    h1 = _silu(u * w1_ref[...] + b1_ref[...])
    gid = i * BLK + lax.broadcasted_iota(jnp.int32, (BLK, 1), 0)
    p_ref[...] = jnp.where(gid < N, dinv * h1, 0.0)


def _stage3_body(t_ref, p_ref, dinv_ref, bat_ref, w2_ref, b2_ref,
                 wl1_ref, bl1_ref, wl2_ref, bl2_ref, out_ref, gacc, cacc):
    i = pl.program_id(0)

    @pl.when(i == 0)
    def _():
        gacc[...] = jnp.zeros_like(gacc)
        cacc[...] = jnp.zeros_like(cacc)

    agg = dinv_ref[...] * (t_ref[...] + p_ref[...])
    out2 = jnp.dot(agg, w2_ref[...],
                   preferred_element_type=jnp.float32) + b2_ref[...]
    h2 = _silu(out2)
    oh = (bat_ref[...] ==
          lax.broadcasted_iota(jnp.int32, (1, G), 1)).astype(jnp.float32)
    gacc[...] += lax.dot_general(oh, h2, (((0,), (0,)), ((), ())),
                                 preferred_element_type=jnp.float32)
    cacc[...] += lax.dot_general(oh, jnp.ones((BLK, 1), jnp.float32),
                                 (((0,), (0,)), ((), ())),
                                 preferred_element_type=jnp.float32)

    @pl.when(i == NBLK - 1)
    def _():
        g = gacc[...] / jnp.maximum(cacc[...], 1.0)
        z1 = _silu(jnp.dot(g, wl1_ref[...],
                           preferred_element_type=jnp.float32) + bl1_ref[...])
        out_ref[...] = jnp.dot(z1, wl2_ref[...],
                               preferred_element_type=jnp.float32) + bl2_ref[...]


def _stage1(degp_t, x1):
    return pl.pallas_call(
        _stage1_body,
        grid=(NBLK,),
        in_specs=[
            pl.BlockSpec((BLK, NW), lambda i: (i, 0)),
            pl.BlockSpec((BLK, 1), lambda i: (i, 0)),
        ],
        out_specs=[
            pl.BlockSpec((BLK, 1), lambda i: (i, 0)),
            pl.BlockSpec((BLK, 1), lambda i: (i, 0)),
        ],
        out_shape=[
            jax.ShapeDtypeStruct((NPAD, 1), jnp.float32),
            jax.ShapeDtypeStruct((NPAD, 1), jnp.float32),
        ],
    )(degp_t, x1)


def _stage2(up_t, qc, dinvc, w1p, b1p):
    return pl.pallas_call(
        _stage2_body,
        grid=(NBLK,),
        in_specs=[
            pl.BlockSpec((BLK, NW), lambda i: (i, 0)),
            pl.BlockSpec((BLK, 1), lambda i: (i, 0)),
            pl.BlockSpec((BLK, 1), lambda i: (i, 0)),
            pl.BlockSpec((1, F1), lambda i: (0, 0)),
            pl.BlockSpec((1, F1), lambda i: (0, 0)),
        ],
        out_specs=pl.BlockSpec((BLK, F1), lambda i: (i, 0)),
        out_shape=jax.ShapeDtypeStruct((NPAD, F1), jnp.float32),
    )(up_t, qc, dinvc, w1p, b1p)


def _stage3(t, p, dinvc, batc, w2p, b2p, wl1p, bl1p, wl2p, bl2p):
    return pl.pallas_call(
        _stage3_body,
        grid=(NBLK,),
        in_specs=[
            pl.BlockSpec((BLK, F1), lambda i: (i, 0)),
            pl.BlockSpec((BLK, F1), lambda i: (i, 0)),
            pl.BlockSpec((BLK, 1), lambda i: (i, 0)),
            pl.BlockSpec((BLK, 1), lambda i: (i, 0)),
            pl.BlockSpec((F1, F2), lambda i: (0, 0)),
            pl.BlockSpec((1, F2), lambda i: (0, 0)),
            pl.BlockSpec((F2, 128), lambda i: (0, 0)),
            pl.BlockSpec((1, 128), lambda i: (0, 0)),
            pl.BlockSpec((128, 128), lambda i: (0, 0)),
            pl.BlockSpec((1, 128), lambda i: (0, 0)),
        ],
        out_specs=pl.BlockSpec((G, 128), lambda i: (0, 0)),
        out_shape=jax.ShapeDtypeStruct((G, 128), jnp.float32),
        scratch_shapes=[
            pltpu.VMEM((G, F2), jnp.float32),
            pltpu.VMEM((G, 1), jnp.float32),
        ],
    )(t, p, dinvc, batc, w2p, b2p, wl1p, bl1p, wl2p, bl2p)


def kernel(x, edge_index, batch, W1, b1, W2, b2, Wl1, bl1, Wl2, bl2):
    f32 = jnp.float32
    src = jnp.pad(edge_index[0], (0, EPAD - E), constant_values=N)
    dst = jnp.pad(edge_index[1], (0, EPAD - E), constant_values=N)
    x1 = jnp.pad(x.astype(f32), ((0, NPAD - N), (0, 0)))
    batc = jnp.pad(batch, (0, NPAD - N), constant_values=G)[:, None]

    w1p = jnp.pad(W1.astype(f32), ((0, 0), (0, F1 - 100)))
    b1p = jnp.pad(b1.astype(f32), (0, F1 - 100))[None, :]
    w2p = jnp.pad(W2.astype(f32), ((0, F1 - 100), (0, F2 - 200)))
    b2p = jnp.pad(b2.astype(f32), (0, F2 - 200))[None, :]
    wl1p = jnp.pad(Wl1.astype(f32), ((0, F2 - 200), (0, 128 - 100)))
    bl1p = jnp.pad(bl1.astype(f32), (0, 128 - 100))[None, :]
    wl2p = jnp.pad(Wl2.astype(f32), ((0, 128 - 100), (0, 127)))
    bl2p = jnp.pad(bl2.astype(f32), (0, 127))[None, :]

    degp = _pass_deg()(dst)
    dinvc, qc = _stage1(degp.T, x1)
    q1 = qc[:, 0]
    urawp = _pass_u()(src, dst, q1)
    p = _stage2(urawp.T, qc, dinvc, w1p, b1p)
    t = _pass_t()(src, dst, p)
    out = _stage3(t, p, dinvc, batc, w2p, b2p, wl1p, bl1p, wl2p, bl2p)
    return out[:, 0]
